# baseline (speedup 1.0000x reference)
import functools

import jax
import jax.numpy as jnp
from jax import lax
from jax.experimental import pallas as pl
from jax.experimental.pallas import tpu as pltpu
from jax.experimental.pallas import tpu_sc as plsc

VOCAB = 1000000
EMBED_DIM = 64
BATCH = 4096
_L = 16


def _sc_gather_pair_t(target, context, in_flat, out_flat):
    info = plsc.get_sparse_core_info()
    nw = info.num_cores * info.num_subcores
    bw = BATCH // nw
    mesh = plsc.VectorSubcoreMesh(core_axis_name="c", subcore_axis_name="s")

    @functools.partial(
        pl.kernel,
        out_type=(
            jax.ShapeDtypeStruct((EMBED_DIM, BATCH), jnp.float32),
            jax.ShapeDtypeStruct((EMBED_DIM, BATCH), jnp.float32),
        ),
        mesh=mesh,
        compiler_params=pltpu.CompilerParams(needs_layout_passes=False),
        scratch_types=[
            pltpu.VMEM((bw,), jnp.int32),
            pltpu.VMEM((bw,), jnp.int32),
            pltpu.VMEM((EMBED_DIM, bw), jnp.float32),
            pltpu.VMEM((EMBED_DIM, bw), jnp.float32),
            pltpu.SemaphoreType.DMA,
            pltpu.SemaphoreType.DMA,
        ],
    )
    def gather_kernel(tgt_hbm, ctx_hbm, in_hbm, out_hbm, tgt_t_hbm, ctx_t_hbm,
                      idx_v, idxd_v, cols_t, cols_c, sem_t, sem_c):
        wid = lax.axis_index("s") * info.num_cores + lax.axis_index("c")
        base = wid * bw

        def one_table(idx_hbm, tab, cols, sem):
            pltpu.sync_copy(idx_hbm.at[pl.ds(base, bw)], idx_v)

            def d_body(d, carry):
                off = d * VOCAB
                for g in range(bw // _L):
                    sl = pl.ds(g * _L, _L)
                    idxd_v[sl] = idx_v[sl] + off
                pltpu.async_copy(tab.at[idxd_v], cols.at[d], sem)
                return carry

            lax.fori_loop(0, EMBED_DIM, d_body, 0)

        def drain(out_hbm_slice, cols, sem):
            pltpu.make_async_copy(out_hbm_slice, cols, sem).wait()

        one_table(tgt_hbm, in_hbm, cols_t, sem_t)
        one_table(ctx_hbm, out_hbm, cols_c, sem_c)
        drain(tgt_t_hbm.at[:, pl.ds(base, bw)], cols_t, sem_t)
        drain(ctx_t_hbm.at[:, pl.ds(base, bw)], cols_c, sem_c)
        pltpu.sync_copy(cols_t, tgt_t_hbm.at[:, pl.ds(base, bw)])
        pltpu.sync_copy(cols_c, ctx_t_hbm.at[:, pl.ds(base, bw)])

    return gather_kernel(target, context, in_flat, out_flat)


def _scores_matmul_t(tgt_t, ctx_t):
    bm = 256

    def mm(a_ref, b_ref, o_ref):
        a16 = a_ref[...].astype(jnp.bfloat16)
        b16 = b_ref[...].astype(jnp.bfloat16)
        o_ref[...] = lax.dot_general(
            a16, b16,
            dimension_numbers=(((0,), (0,)), ((), ())),
            preferred_element_type=jnp.float32,
        )

    return pl.pallas_call(
        mm,
        grid=(BATCH // bm,),
        in_specs=[
            pl.BlockSpec((EMBED_DIM, bm), lambda i: (0, i)),
            pl.BlockSpec((EMBED_DIM, BATCH), lambda i: (0, 0)),
        ],
        out_specs=pl.BlockSpec((bm, BATCH), lambda i: (i, 0)),
        out_shape=jax.ShapeDtypeStruct((BATCH, BATCH), jnp.float32),
    )(tgt_t, ctx_t)


def kernel(target, context, in_embed_weight, out_embed_weight):
    in_flat = in_embed_weight.T.reshape(-1)
    out_flat = out_embed_weight.T.reshape(-1)
    tgt_t, ctx_t = _sc_gather_pair_t(target, context, in_flat, out_flat)
    return _scores_matmul_t(tgt_t, ctx_t)


# speedup vs baseline: 10.0673x; 10.0673x over previous
import functools

import jax
import jax.numpy as jnp
from jax import lax
from jax.experimental import pallas as pl
from jax.experimental.pallas import tpu as pltpu
from jax.experimental.pallas import tpu_sc as plsc

VOCAB = 1000000
EMBED_DIM = 64
BATCH = 4096


def _sc_gather(context, out_tab):
    info = plsc.get_sparse_core_info()
    nw = info.num_cores * info.num_subcores
    bw = BATCH // nw
    mesh = plsc.VectorSubcoreMesh(core_axis_name="c", subcore_axis_name="s")

    @functools.partial(
        pl.kernel,
        out_type=jax.ShapeDtypeStruct((BATCH, EMBED_DIM), jnp.float32),
        mesh=mesh,
        compiler_params=pltpu.CompilerParams(use_tc_tiling_on_sc=False),
        scratch_types=[
            pltpu.VMEM((bw,), jnp.int32),
            pltpu.VMEM((bw, EMBED_DIM), jnp.float32),
            pltpu.SemaphoreType.DMA,
        ],
    )
    def gather_kernel(ctx_hbm, tab_hbm, rows_hbm, idx_v, rows_v, sem):
        wid = lax.axis_index("s") * info.num_cores + lax.axis_index("c")
        base = wid * bw
        pltpu.sync_copy(ctx_hbm.at[pl.ds(base, bw)], idx_v)
        pltpu.async_copy(tab_hbm.at[idx_v], rows_v, sem).wait()
        pltpu.sync_copy(rows_v, rows_hbm.at[pl.ds(base, bw)])

    return gather_kernel(context, out_tab)


def _tc_gather(target, in_tab):
    def body(tgt_s, tab_hbm, rows_vmem, sem):
        def row_body(r, carry):
            ri = tgt_s[r]
            pltpu.make_async_copy(
                tab_hbm.at[pl.ds(ri, 1)],
                rows_vmem.at[pl.ds(r, 1)], sem).start()
            return carry

        lax.fori_loop(0, BATCH, row_body, 0)
        pltpu.make_async_copy(
            tab_hbm.at[pl.ds(0, BATCH)], rows_vmem, sem).wait()

    return pl.pallas_call(
        body,
        in_specs=[
            pl.BlockSpec(memory_space=pltpu.SMEM),
            pl.BlockSpec(memory_space=pl.ANY),
        ],
        out_specs=pl.BlockSpec(memory_space=pltpu.VMEM),
        out_shape=jax.ShapeDtypeStruct((BATCH, EMBED_DIM), jnp.float32),
        scratch_shapes=[pltpu.SemaphoreType.DMA],
    )(target, in_tab)


def _scores_matmul(tgt_rows, ctx_rows):
    bm = 256

    def mm(a_ref, b_ref, o_ref):
        a16 = a_ref[...].astype(jnp.bfloat16)
        b16 = b_ref[...].astype(jnp.bfloat16)
        o_ref[...] = lax.dot_general(
            a16, b16,
            dimension_numbers=(((1,), (1,)), ((), ())),
            preferred_element_type=jnp.float32,
        )

    return pl.pallas_call(
        mm,
        grid=(BATCH // bm,),
        in_specs=[
            pl.BlockSpec((bm, EMBED_DIM), lambda i: (i, 0)),
            pl.BlockSpec((BATCH, EMBED_DIM), lambda i: (0, 0)),
        ],
        out_specs=pl.BlockSpec((bm, BATCH), lambda i: (i, 0)),
        out_shape=jax.ShapeDtypeStruct((BATCH, BATCH), jnp.float32),
    )(tgt_rows, ctx_rows)


def kernel(target, context, in_embed_weight, out_embed_weight):
    ctx_rows = _sc_gather(context, out_embed_weight)
    tgt_rows = _tc_gather(target, in_embed_weight)
    return _scores_matmul(tgt_rows, ctx_rows)


# speedup vs baseline: 16.6730x; 1.6562x over previous
import jax
import jax.numpy as jnp
from jax import lax
from jax.experimental import pallas as pl
from jax.experimental.pallas import tpu as pltpu

VOCAB = 1000000
EMBED_DIM = 64
BATCH = 4096


def _transpose_table(wt):
    bk = 8192

    def tr(a_ref, o_ref):
        o_ref[...] = a_ref[...].T

    return pl.pallas_call(
        tr,
        grid=((VOCAB + bk - 1) // bk,),
        in_specs=[pl.BlockSpec((EMBED_DIM, bk), lambda i: (0, i))],
        out_specs=pl.BlockSpec((bk, EMBED_DIM), lambda i: (i, 0)),
        out_shape=jax.ShapeDtypeStruct((VOCAB, EMBED_DIM), jnp.float32),
    )(wt)


def _gather_rows(target, context, in_tab, out_tab):
    def body(tgt_s, ctx_s, in_hbm, out_hbm, tgt_rows, ctx_rows, sem_t, sem_c):
        def one_table(idx_s, tab, rows, sem):
            def row_body(r, carry):
                ri = idx_s[r]
                pltpu.make_async_copy(
                    tab.at[pl.ds(ri, 1)], rows.at[pl.ds(r, 1)], sem).start()
                return carry

            lax.fori_loop(0, BATCH, row_body, 0)

        one_table(tgt_s, in_hbm, tgt_rows, sem_t)
        one_table(ctx_s, out_hbm, ctx_rows, sem_c)
        pltpu.make_async_copy(
            in_hbm.at[pl.ds(0, BATCH)], tgt_rows, sem_t).wait()
        pltpu.make_async_copy(
            out_hbm.at[pl.ds(0, BATCH)], ctx_rows, sem_c).wait()

    return pl.pallas_call(
        body,
        in_specs=[
            pl.BlockSpec(memory_space=pltpu.SMEM),
            pl.BlockSpec(memory_space=pltpu.SMEM),
            pl.BlockSpec(memory_space=pl.ANY),
            pl.BlockSpec(memory_space=pl.ANY),
        ],
        out_specs=[
            pl.BlockSpec(memory_space=pltpu.VMEM),
            pl.BlockSpec(memory_space=pltpu.VMEM),
        ],
        out_shape=[
            jax.ShapeDtypeStruct((BATCH, EMBED_DIM), jnp.float32),
            jax.ShapeDtypeStruct((BATCH, EMBED_DIM), jnp.float32),
        ],
        scratch_shapes=[pltpu.SemaphoreType.DMA, pltpu.SemaphoreType.DMA],
    )(target, context, in_tab, out_tab)


def _scores_matmul(tgt_rows, ctx_rows):
    bm = 256

    def mm(a_ref, b_ref, o_ref):
        a16 = a_ref[...].astype(jnp.bfloat16)
        b16 = b_ref[...].astype(jnp.bfloat16)
        o_ref[...] = lax.dot_general(
            a16, b16,
            dimension_numbers=(((1,), (1,)), ((), ())),
            preferred_element_type=jnp.float32,
        )

    return pl.pallas_call(
        mm,
        grid=(BATCH // bm,),
        in_specs=[
            pl.BlockSpec((bm, EMBED_DIM), lambda i: (i, 0)),
            pl.BlockSpec((BATCH, EMBED_DIM), lambda i: (0, 0)),
        ],
        out_specs=pl.BlockSpec((bm, BATCH), lambda i: (i, 0)),
        out_shape=jax.ShapeDtypeStruct((BATCH, BATCH), jnp.float32),
    )(tgt_rows, ctx_rows)


def kernel(target, context, in_embed_weight, out_embed_weight):
    in_tab = _transpose_table(in_embed_weight.T)
    out_tab = _transpose_table(out_embed_weight.T)
    tgt_rows, ctx_rows = _gather_rows(target, context, in_tab, out_tab)
    return _scores_matmul(tgt_rows, ctx_rows)


# speedup vs baseline: 17.8062x; 1.0680x over previous
import jax
import jax.numpy as jnp
from jax import lax
from jax.experimental import pallas as pl
from jax.experimental.pallas import tpu as pltpu

VOCAB = 1000000
EMBED_DIM = 64
BATCH = 4096


def _transpose_table(wt):
    bk = 16384

    def tr(a_ref, o_ref):
        o_ref[...] = a_ref[...].T

    return pl.pallas_call(
        tr,
        grid=((VOCAB + bk - 1) // bk,),
        in_specs=[pl.BlockSpec((EMBED_DIM, bk), lambda i: (0, i))],
        out_specs=pl.BlockSpec((bk, EMBED_DIM), lambda i: (i, 0)),
        out_shape=jax.ShapeDtypeStruct((VOCAB, EMBED_DIM), jnp.float32),
    )(wt)


def _gather_rows(target, context, in_tab, out_tab):
    def body(tgt_s, ctx_s, in_hbm, out_hbm, tgt_rows, ctx_rows, sem_t, sem_c):
        def one_table(idx_s, tab, rows, sem):
            def row_body(r, carry):
                ri = idx_s[r]
                pltpu.make_async_copy(
                    tab.at[pl.ds(ri, 1)], rows.at[pl.ds(r, 1)], sem).start()
                return carry

            lax.fori_loop(0, BATCH, row_body, 0)

        one_table(tgt_s, in_hbm, tgt_rows, sem_t)
        one_table(ctx_s, out_hbm, ctx_rows, sem_c)
        pltpu.make_async_copy(
            in_hbm.at[pl.ds(0, BATCH)], tgt_rows, sem_t).wait()
        pltpu.make_async_copy(
            out_hbm.at[pl.ds(0, BATCH)], ctx_rows, sem_c).wait()

    return pl.pallas_call(
        body,
        in_specs=[
            pl.BlockSpec(memory_space=pltpu.SMEM),
            pl.BlockSpec(memory_space=pltpu.SMEM),
            pl.BlockSpec(memory_space=pl.ANY),
            pl.BlockSpec(memory_space=pl.ANY),
        ],
        out_specs=[
            pl.BlockSpec(memory_space=pltpu.VMEM),
            pl.BlockSpec(memory_space=pltpu.VMEM),
        ],
        out_shape=[
            jax.ShapeDtypeStruct((BATCH, EMBED_DIM), jnp.float32),
            jax.ShapeDtypeStruct((BATCH, EMBED_DIM), jnp.float32),
        ],
        scratch_shapes=[pltpu.SemaphoreType.DMA, pltpu.SemaphoreType.DMA],
    )(target, context, in_tab, out_tab)


def _scores_matmul(tgt_rows, ctx_rows):
    bm = 256

    def mm(a_ref, b_ref, o_ref):
        a16 = a_ref[...].astype(jnp.bfloat16)
        b16 = b_ref[...].astype(jnp.bfloat16)
        o_ref[...] = lax.dot_general(
            a16, b16,
            dimension_numbers=(((1,), (1,)), ((), ())),
            preferred_element_type=jnp.float32,
        )

    return pl.pallas_call(
        mm,
        grid=(BATCH // bm,),
        in_specs=[
            pl.BlockSpec((bm, EMBED_DIM), lambda i: (i, 0)),
            pl.BlockSpec((BATCH, EMBED_DIM), lambda i: (0, 0)),
        ],
        out_specs=pl.BlockSpec((bm, BATCH), lambda i: (i, 0)),
        out_shape=jax.ShapeDtypeStruct((BATCH, BATCH), jnp.float32),
    )(tgt_rows, ctx_rows)


def kernel(target, context, in_embed_weight, out_embed_weight):
    in_tab = _transpose_table(in_embed_weight.T)
    out_tab = _transpose_table(out_embed_weight.T)
    tgt_rows, ctx_rows = _gather_rows(target, context, in_tab, out_tab)
    return _scores_matmul(tgt_rows, ctx_rows)
